# baseline (speedup 1.0000x reference)
import functools

import jax
import jax.numpy as jnp
from jax import lax
from jax.experimental import pallas as pl
from jax.experimental.pallas import tpu as pltpu
from jax.experimental.pallas import tpu_sc as plsc

N_NODES = 10000
N_EDGES = 320000
EPS = 1e-8
ALPHA_DT = 0.6 / (1000.0 * 4186.0) * 1e-05

LANES = 16
NC = 2
NS = 16
NW = NC * NS
NPAD = 10240
EPW = N_EDGES // NW
SLICE = NPAD // NS

_mesh = plsc.VectorSubcoreMesh(
    core_axis_name="c", subcore_axis_name="s", num_cores=NC, num_subcores=NS
)
_sc_params = pltpu.CompilerParams(needs_layout_passes=False)


def _zero_refs(refs):
    z = jnp.zeros((LANES,), jnp.float32)

    @plsc.parallel_loop(0, NPAD // LANES, unroll=4)
    def _(i):
        for r in refs:
            r[pl.ds(i * LANES, LANES)] = z


def _reduce_rows(red, redout):
    @plsc.parallel_loop(0, SLICE // LANES, unroll=2)
    def _(j):
        o = j * LANES
        acc = red[pl.ds(o, LANES)]
        for r in range(1, NS):
            acc = acc + red[pl.ds(r * SLICE + o, LANES)]
        redout[pl.ds(o, LANES)] = acc


XF = 128
XCHUNK = 64
TSL = 640
EOFF = NPAD


@functools.partial(
    pl.kernel,
    out_type=[jax.ShapeDtypeStruct((NC * NPAD,), jnp.float32)] * 4,
    mesh=_mesh,
    scratch_types=[
        pltpu.VMEM((NPAD,), jnp.float32),
        pltpu.VMEM((NPAD,), jnp.float32),
        pltpu.VMEM((NPAD,), jnp.float32),
        pltpu.VMEM((NPAD,), jnp.float32),
        pltpu.VMEM((NPAD,), jnp.float32),
        pltpu.VMEM((NPAD,), jnp.float32),
        pltpu.VMEM((NPAD,), jnp.float32),
        pltpu.VMEM((NPAD,), jnp.float32),
        pltpu.VMEM((2 * NPAD,), jnp.int32),
        pltpu.VMEM((TSL,), jnp.float32),
        pltpu.VMEM((NS * SLICE,), jnp.float32),
        pltpu.VMEM((SLICE,), jnp.float32),
        pltpu.SemaphoreType.DMA,
        pltpu.SemaphoreType.DMA,
        pltpu.VMEM_SHARED((NPAD,), jnp.float32),
        pltpu.VMEM_SHARED((NS * NPAD,), jnp.float32),
    ],
    compiler_params=_sc_params,
)
def _grad_kernel(xi_h, px_h, py_h, pz_h, edges_h,
                 onx, ony, onz, ocnt,
                 tT, tpx, tpy, tpz, ax, ay, az, ac,
                 buf, tcol, red, redout, sem_pos, sem_x, shT, sh):
    cid = lax.axis_index("c")
    sid = lax.axis_index("s")
    wid = cid * NS + sid
    ebase = wid * EPW

    pos_descs = [
        pltpu.async_copy(px_h, tpx.at[pl.ds(0, N_NODES)], sem_pos),
        pltpu.async_copy(py_h, tpy.at[pl.ds(0, N_NODES)], sem_pos),
        pltpu.async_copy(pz_h, tpz.at[pl.ds(0, N_NODES)], sem_pos),
    ]

    col_idx = XF * lax.iota(jnp.int32, LANES) + 3
    nodebase = sid * TSL
    HALF = XCHUNK * XF

    def _extract_chunks(n_chunks, rows_tail):
        sizes = [XCHUNK] * n_chunks + ([rows_tail] if rows_tail else [])

        def _start(c):
            return pltpu.async_copy(
                xi_h.at[pl.ds((nodebase + c * XCHUNK) * XF, sizes[c] * XF)],
                buf.at[pl.ds((c % 2) * HALF, sizes[c] * XF)], sem_x)

        descs = [_start(0)]
        for c in range(len(sizes)):
            descs[c].wait()
            if c + 1 < len(sizes):
                descs.append(_start(c + 1))
            half = (c % 2) * HALF
            for j in range(sizes[c] // LANES):
                v = plsc.load_gather(buf, [col_idx + (half + j * LANES * XF)])
                tcol[pl.ds(c * XCHUNK + j * LANES, LANES)] = plsc.bitcast(
                    v, jnp.float32)

    TAILR = N_NODES - (NS - 1) * TSL

    @pl.when(sid < NS - 1)
    def _():
        _extract_chunks(TSL // XCHUNK, 0)
        pltpu.sync_copy(tcol, shT.at[pl.ds(nodebase, TSL)])

    @pl.when(sid == NS - 1)
    def _():
        _extract_chunks(TAILR // XCHUNK, TAILR % XCHUNK)
        pltpu.sync_copy(tcol.at[pl.ds(0, TAILR)],
                        shT.at[pl.ds(nodebase, TAILR)])

    pltpu.sync_copy(edges_h.at[pl.ds(ebase, EPW)], buf.at[pl.ds(0, EPW)])
    pltpu.sync_copy(edges_h.at[pl.ds(N_EDGES + ebase, EPW)],
                    buf.at[pl.ds(EOFF, EPW)])
    _zero_refs([ax, ay, az, ac])
    for dsc in pos_descs:
        dsc.wait()
    plsc.subcore_barrier()
    pltpu.sync_copy(shT, tT)
    ones = jnp.full((LANES,), 1.0, jnp.float32)

    @plsc.parallel_loop(0, EPW // LANES, unroll=4)
    def _(i):
        o = pl.ds(i * LANES, LANES)
        s = buf[o]
        d = buf[pl.ds(EOFF + i * LANES, LANES)]
        ts = plsc.load_gather(tT, [s])
        td = plsc.load_gather(tT, [d])
        pxs = plsc.load_gather(tpx, [s])
        pxd = plsc.load_gather(tpx, [d])
        pys = plsc.load_gather(tpy, [s])
        pyd = plsc.load_gather(tpy, [d])
        pzs = plsc.load_gather(tpz, [s])
        pzd = plsc.load_gather(tpz, [d])
        dx = pxd - pxs
        dy = pyd - pys
        dz = pzd - pzs
        dist2 = dx * dx + dy * dy + dz * dz + EPS
        w = (td - ts) / dist2
        plsc.addupdate_scatter(ax, [d], w * dx)
        plsc.addupdate_scatter(ay, [d], w * dy)
        plsc.addupdate_scatter(az, [d], w * dz)
        plsc.addupdate_scatter(ac, [d], ones)

    nbase = sid * SLICE
    for k, (acc, out) in enumerate([(ax, onx), (ay, ony), (az, onz), (ac, ocnt)]):
        if k > 0:
            plsc.subcore_barrier()
        pltpu.sync_copy(acc, sh.at[pl.ds(sid * NPAD, NPAD)])
        plsc.subcore_barrier()
        for r in range(NS):
            pltpu.sync_copy(sh.at[pl.ds(r * NPAD + nbase, SLICE)],
                            red.at[pl.ds(r * SLICE, SLICE)])
        _reduce_rows(red, redout)
        pltpu.sync_copy(redout, out.at[pl.ds(cid * NPAD + nbase, SLICE)])


@functools.partial(
    pl.kernel,
    out_type=jax.ShapeDtypeStruct((NC * NPAD,), jnp.float32),
    mesh=_mesh,
    scratch_types=[
        pltpu.VMEM((NPAD,), jnp.float32),
        pltpu.VMEM((NPAD,), jnp.float32),
        pltpu.VMEM((NPAD,), jnp.float32),
        pltpu.VMEM((NPAD,), jnp.float32),
        pltpu.VMEM((NPAD,), jnp.float32),
        pltpu.VMEM((NPAD,), jnp.float32),
        pltpu.VMEM((NPAD,), jnp.float32),
        pltpu.VMEM((EPW,), jnp.int32),
        pltpu.VMEM((EPW,), jnp.int32),
        pltpu.VMEM((NS * SLICE,), jnp.float32),
        pltpu.VMEM((SLICE,), jnp.float32),
        pltpu.VMEM((NC * SLICE,), jnp.float32),
        pltpu.VMEM((SLICE,), jnp.float32),
        pltpu.VMEM_SHARED((3 * NPAD,), jnp.float32),
        pltpu.VMEM_SHARED((NS * NPAD,), jnp.float32),
    ],
    compiler_params=_sc_params,
)
def _div_kernel(pnx, pny, pnz, pcnt, px_h, py_h, pz_h, edges_h,
                odiv,
                tpx, tpy, tpz, tgx, tgy, tgz, adiv,
                esrc, edst, red, redout, pbuf, ccnt, shg, sh):
    cid = lax.axis_index("c")
    sid = lax.axis_index("s")
    wid = cid * NS + sid
    ebase = wid * EPW
    nbase = sid * SLICE

    pltpu.sync_copy(px_h, tpx.at[pl.ds(0, N_NODES)])
    pltpu.sync_copy(py_h, tpy.at[pl.ds(0, N_NODES)])
    pltpu.sync_copy(pz_h, tpz.at[pl.ds(0, N_NODES)])
    pltpu.sync_copy(edges_h.at[pl.ds(ebase, EPW)], esrc)
    pltpu.sync_copy(edges_h.at[pl.ds(N_EDGES + ebase, EPW)], edst)

    pltpu.sync_copy(pcnt.at[pl.ds(nbase, SLICE)], pbuf.at[pl.ds(0, SLICE)])
    pltpu.sync_copy(pcnt.at[pl.ds(NPAD + nbase, SLICE)],
                    pbuf.at[pl.ds(SLICE, SLICE)])

    def cnt_body(j, _):
        o = j * LANES
        ccnt[pl.ds(o, LANES)] = (
            pbuf[pl.ds(o, LANES)] + pbuf[pl.ds(SLICE + o, LANES)] + EPS
        )
        return 0

    lax.fori_loop(0, SLICE // LANES, cnt_body, 0)

    for k, pn in enumerate([pnx, pny, pnz]):
        pltpu.sync_copy(pn.at[pl.ds(nbase, SLICE)], pbuf.at[pl.ds(0, SLICE)])
        pltpu.sync_copy(pn.at[pl.ds(NPAD + nbase, SLICE)],
                        pbuf.at[pl.ds(SLICE, SLICE)])

        def g_body(j, _):
            o = j * LANES
            redout[pl.ds(o, LANES)] = (
                pbuf[pl.ds(o, LANES)] + pbuf[pl.ds(SLICE + o, LANES)]
            ) / ccnt[pl.ds(o, LANES)]
            return 0

        lax.fori_loop(0, SLICE // LANES, g_body, 0)
        pltpu.sync_copy(redout, shg.at[pl.ds(k * NPAD + nbase, SLICE)])

    plsc.subcore_barrier()
    pltpu.sync_copy(shg.at[pl.ds(0, NPAD)], tgx)
    pltpu.sync_copy(shg.at[pl.ds(NPAD, NPAD)], tgy)
    pltpu.sync_copy(shg.at[pl.ds(2 * NPAD, NPAD)], tgz)

    _zero_refs([adiv])

    @plsc.parallel_loop(0, EPW // LANES, unroll=4)
    def _(i):
        o = pl.ds(i * LANES, LANES)
        s = esrc[o]
        d = edst[o]
        pxs = plsc.load_gather(tpx, [s])
        pxd = plsc.load_gather(tpx, [d])
        pys = plsc.load_gather(tpy, [s])
        pyd = plsc.load_gather(tpy, [d])
        pzs = plsc.load_gather(tpz, [s])
        pzd = plsc.load_gather(tpz, [d])
        gxs = plsc.load_gather(tgx, [s])
        gxd = plsc.load_gather(tgx, [d])
        gys = plsc.load_gather(tgy, [s])
        gyd = plsc.load_gather(tgy, [d])
        gzs = plsc.load_gather(tgz, [s])
        gzd = plsc.load_gather(tgz, [d])
        dx = pxd - pxs
        dy = pyd - pys
        dz = pzd - pzs
        dist2 = dx * dx + dy * dy + dz * dz + EPS
        dive = ((gxd - gxs) * dx + (gyd - gys) * dy + (gzd - gzs) * dz) / dist2
        plsc.addupdate_scatter(adiv, [d], dive)

    pltpu.sync_copy(adiv, sh.at[pl.ds(sid * NPAD, NPAD)])
    plsc.subcore_barrier()
    for r in range(NS):
        pltpu.sync_copy(sh.at[pl.ds(r * NPAD + nbase, SLICE)],
                        red.at[pl.ds(r * SLICE, SLICE)])
    _reduce_rows(red, redout)
    pltpu.sync_copy(redout, odiv.at[pl.ds(cid * NPAD + nbase, SLICE)])


def _loss_body(div_ref, cnt_ref, dtp_ref, out_ref):
    d = div_ref[0:1, :] + div_ref[1:2, :]
    c = cnt_ref[0:1, :] + cnt_ref[1:2, :]
    lap = d / (c + EPS)
    diff = dtp_ref[...] - ALPHA_DT * lap
    out_ref[...] = jnp.sum(diff * diff, keepdims=True) * (1.0 / N_NODES)


_loss_call = pl.pallas_call(
    _loss_body,
    out_shape=jax.ShapeDtypeStruct((1, 1), jnp.float32),
)


def kernel(pred, target, x, pos, edge_index, edge_attr):
    xi = jax.lax.bitcast_convert_type(x, jnp.int32).reshape(N_NODES * XF)
    px = pos[:, 0]
    py = pos[:, 1]
    pz = pos[:, 2]
    edges = edge_index.astype(jnp.int32).reshape(2 * N_EDGES)
    dtp = jnp.pad(pred[:, 0], (0, NPAD - N_NODES)).reshape(1, NPAD)

    nx, ny, nz, cnt = _grad_kernel(xi, px, py, pz, edges)
    divp = _div_kernel(nx, ny, nz, cnt, px, py, pz, edges)
    loss = _loss_call(divp.reshape(NC, NPAD), cnt.reshape(NC, NPAD), dtp)
    return loss[0, 0]


# speedup vs baseline: 1.1436x; 1.1436x over previous
import functools

import jax
import jax.numpy as jnp
from jax import lax
from jax.experimental import pallas as pl
from jax.experimental.pallas import tpu as pltpu
from jax.experimental.pallas import tpu_sc as plsc

N_NODES = 10000
N_EDGES = 320000
EPS = 1e-8
ALPHA_DT = 0.6 / (1000.0 * 4186.0) * 1e-05

LANES = 16
NC = 2
NS = 16
NW = NC * NS
NPAD = 10240
EPW = N_EDGES // NW
SLICE = NPAD // NS

_mesh = plsc.VectorSubcoreMesh(
    core_axis_name="c", subcore_axis_name="s", num_cores=NC, num_subcores=NS
)
_sc_params = pltpu.CompilerParams(needs_layout_passes=False)


def _zero_refs(refs):
    z = jnp.zeros((LANES,), jnp.float32)

    @plsc.parallel_loop(0, NPAD // LANES, unroll=4)
    def _(i):
        for r in refs:
            r[pl.ds(i * LANES, LANES)] = z


def _reduce_rows(red, redout):
    @plsc.parallel_loop(0, SLICE // LANES, unroll=2)
    def _(j):
        o = j * LANES
        acc = red[pl.ds(o, LANES)]
        for r in range(1, NS):
            acc = acc + red[pl.ds(r * SLICE + o, LANES)]
        redout[pl.ds(o, LANES)] = acc


XF = 128
XCHUNK = 64
TSL = 640
EOFF = NPAD


@functools.partial(
    pl.kernel,
    out_type=[jax.ShapeDtypeStruct((NC * NPAD,), jnp.float32)] * 4,
    mesh=_mesh,
    scratch_types=[
        pltpu.VMEM((NPAD,), jnp.float32),
        pltpu.VMEM((NPAD,), jnp.float32),
        pltpu.VMEM((NPAD,), jnp.float32),
        pltpu.VMEM((NPAD,), jnp.float32),
        pltpu.VMEM((NPAD,), jnp.float32),
        pltpu.VMEM((NPAD,), jnp.float32),
        pltpu.VMEM((NPAD,), jnp.float32),
        pltpu.VMEM((NPAD,), jnp.float32),
        pltpu.VMEM((2 * NPAD,), jnp.int32),
        pltpu.VMEM((TSL,), jnp.float32),
        pltpu.VMEM((NS * SLICE,), jnp.float32),
        pltpu.VMEM((SLICE,), jnp.float32),
        pltpu.SemaphoreType.DMA,
        pltpu.SemaphoreType.DMA,
        pltpu.VMEM_SHARED((NPAD,), jnp.float32),
        pltpu.VMEM_SHARED((NS * NPAD,), jnp.float32),
    ],
    compiler_params=_sc_params,
)
def _grad_kernel(xi_h, px_h, py_h, pz_h, edges_h,
                 onx, ony, onz, ocnt,
                 tT, tpx, tpy, tpz, ax, ay, az, ac,
                 buf, tcol, red, redout, sem_pos, sem_x, shT, sh):
    cid = lax.axis_index("c")
    sid = lax.axis_index("s")
    wid = cid * NS + sid
    ebase = wid * EPW

    pos_descs = [
        pltpu.async_copy(px_h, tpx.at[pl.ds(0, N_NODES)], sem_pos),
        pltpu.async_copy(py_h, tpy.at[pl.ds(0, N_NODES)], sem_pos),
        pltpu.async_copy(pz_h, tpz.at[pl.ds(0, N_NODES)], sem_pos),
    ]

    col_idx = XF * lax.iota(jnp.int32, LANES) + 3
    nodebase = sid * TSL
    HALF = XCHUNK * XF

    def _extract_chunks(n_chunks, rows_tail):
        sizes = [XCHUNK] * n_chunks + ([rows_tail] if rows_tail else [])

        def _start(c):
            return pltpu.async_copy(
                xi_h.at[pl.ds((nodebase + c * XCHUNK) * XF, sizes[c] * XF)],
                buf.at[pl.ds((c % 2) * HALF, sizes[c] * XF)], sem_x)

        descs = [_start(0)]
        for c in range(len(sizes)):
            descs[c].wait()
            if c + 1 < len(sizes):
                descs.append(_start(c + 1))
            half = (c % 2) * HALF
            for j in range(sizes[c] // LANES):
                v = plsc.load_gather(buf, [col_idx + (half + j * LANES * XF)])
                tcol[pl.ds(c * XCHUNK + j * LANES, LANES)] = plsc.bitcast(
                    v, jnp.float32)

    TAILR = N_NODES - (NS - 1) * TSL

    @pl.when(sid < NS - 1)
    def _():
        _extract_chunks(TSL // XCHUNK, 0)
        pltpu.sync_copy(tcol, shT.at[pl.ds(nodebase, TSL)])

    @pl.when(sid == NS - 1)
    def _():
        _extract_chunks(TAILR // XCHUNK, TAILR % XCHUNK)
        pltpu.sync_copy(tcol.at[pl.ds(0, TAILR)],
                        shT.at[pl.ds(nodebase, TAILR)])

    pltpu.sync_copy(edges_h.at[pl.ds(ebase, EPW)], buf.at[pl.ds(0, EPW)])
    pltpu.sync_copy(edges_h.at[pl.ds(N_EDGES + ebase, EPW)],
                    buf.at[pl.ds(EOFF, EPW)])
    _zero_refs([ax, ay, az, ac])
    for dsc in pos_descs:
        dsc.wait()
    plsc.subcore_barrier()
    pltpu.sync_copy(shT, tT)
    ones = jnp.full((LANES,), 1.0, jnp.float32)

    @plsc.parallel_loop(0, EPW // LANES, unroll=4)
    def _(i):
        o = pl.ds(i * LANES, LANES)
        s = buf[o]
        d = buf[pl.ds(EOFF + i * LANES, LANES)]
        ts = plsc.load_gather(tT, [s])
        td = plsc.load_gather(tT, [d])
        pxs = plsc.load_gather(tpx, [s])
        pxd = plsc.load_gather(tpx, [d])
        pys = plsc.load_gather(tpy, [s])
        pyd = plsc.load_gather(tpy, [d])
        pzs = plsc.load_gather(tpz, [s])
        pzd = plsc.load_gather(tpz, [d])
        dx = pxd - pxs
        dy = pyd - pys
        dz = pzd - pzs
        dist2 = dx * dx + dy * dy + dz * dz + EPS
        w = (td - ts) / dist2
        plsc.addupdate_scatter(ax, [d], w * dx)
        plsc.addupdate_scatter(ay, [d], w * dy)
        plsc.addupdate_scatter(az, [d], w * dz)
        plsc.addupdate_scatter(ac, [d], ones)

    nbase = sid * SLICE
    for k, (acc, out) in enumerate([(ax, onx), (ay, ony), (az, onz), (ac, ocnt)]):
        if k > 0:
            plsc.subcore_barrier()
        pltpu.sync_copy(acc, sh.at[pl.ds(sid * NPAD, NPAD)])
        plsc.subcore_barrier()
        descs = [
            pltpu.async_copy(sh.at[pl.ds(r * NPAD + nbase, SLICE)],
                             red.at[pl.ds(r * SLICE, SLICE)], sem_x)
            for r in range(NS)
        ]
        for dsc in descs:
            dsc.wait()
        _reduce_rows(red, redout)
        pltpu.sync_copy(redout, out.at[pl.ds(cid * NPAD + nbase, SLICE)])


@functools.partial(
    pl.kernel,
    out_type=jax.ShapeDtypeStruct((NC * NPAD,), jnp.float32),
    mesh=_mesh,
    scratch_types=[
        pltpu.VMEM((NPAD,), jnp.float32),
        pltpu.VMEM((NPAD,), jnp.float32),
        pltpu.VMEM((NPAD,), jnp.float32),
        pltpu.VMEM((NPAD,), jnp.float32),
        pltpu.VMEM((NPAD,), jnp.float32),
        pltpu.VMEM((NPAD,), jnp.float32),
        pltpu.VMEM((NPAD,), jnp.float32),
        pltpu.VMEM((EPW,), jnp.int32),
        pltpu.VMEM((EPW,), jnp.int32),
        pltpu.VMEM((NS * SLICE,), jnp.float32),
        pltpu.VMEM((SLICE,), jnp.float32),
        pltpu.VMEM((NC * SLICE,), jnp.float32),
        pltpu.VMEM((SLICE,), jnp.float32),
        pltpu.SemaphoreType.DMA,
        pltpu.VMEM_SHARED((3 * NPAD,), jnp.float32),
        pltpu.VMEM_SHARED((NS * NPAD,), jnp.float32),
    ],
    compiler_params=_sc_params,
)
def _div_kernel(pnx, pny, pnz, pcnt, px_h, py_h, pz_h, edges_h,
                odiv,
                tpx, tpy, tpz, tgx, tgy, tgz, adiv,
                esrc, edst, red, redout, pbuf, ccnt, sem, shg, sh):
    cid = lax.axis_index("c")
    sid = lax.axis_index("s")
    wid = cid * NS + sid
    ebase = wid * EPW
    nbase = sid * SLICE

    stage_descs = [
        pltpu.async_copy(px_h, tpx.at[pl.ds(0, N_NODES)], sem),
        pltpu.async_copy(py_h, tpy.at[pl.ds(0, N_NODES)], sem),
        pltpu.async_copy(pz_h, tpz.at[pl.ds(0, N_NODES)], sem),
        pltpu.async_copy(edges_h.at[pl.ds(ebase, EPW)], esrc, sem),
        pltpu.async_copy(edges_h.at[pl.ds(N_EDGES + ebase, EPW)], edst, sem),
    ]

    pltpu.sync_copy(pcnt.at[pl.ds(nbase, SLICE)], pbuf.at[pl.ds(0, SLICE)])
    pltpu.sync_copy(pcnt.at[pl.ds(NPAD + nbase, SLICE)],
                    pbuf.at[pl.ds(SLICE, SLICE)])

    def cnt_body(j, _):
        o = j * LANES
        ccnt[pl.ds(o, LANES)] = (
            pbuf[pl.ds(o, LANES)] + pbuf[pl.ds(SLICE + o, LANES)] + EPS
        )
        return 0

    lax.fori_loop(0, SLICE // LANES, cnt_body, 0)

    for k, pn in enumerate([pnx, pny, pnz]):
        pltpu.sync_copy(pn.at[pl.ds(nbase, SLICE)], pbuf.at[pl.ds(0, SLICE)])
        pltpu.sync_copy(pn.at[pl.ds(NPAD + nbase, SLICE)],
                        pbuf.at[pl.ds(SLICE, SLICE)])

        def g_body(j, _):
            o = j * LANES
            redout[pl.ds(o, LANES)] = (
                pbuf[pl.ds(o, LANES)] + pbuf[pl.ds(SLICE + o, LANES)]
            ) / ccnt[pl.ds(o, LANES)]
            return 0

        lax.fori_loop(0, SLICE // LANES, g_body, 0)
        pltpu.sync_copy(redout, shg.at[pl.ds(k * NPAD + nbase, SLICE)])

    plsc.subcore_barrier()
    gdescs = [
        pltpu.async_copy(shg.at[pl.ds(0, NPAD)], tgx, sem),
        pltpu.async_copy(shg.at[pl.ds(NPAD, NPAD)], tgy, sem),
        pltpu.async_copy(shg.at[pl.ds(2 * NPAD, NPAD)], tgz, sem),
    ]
    _zero_refs([adiv])
    for dsc in stage_descs:
        dsc.wait()
    for dsc in gdescs:
        dsc.wait()

    @plsc.parallel_loop(0, EPW // LANES, unroll=4)
    def _(i):
        o = pl.ds(i * LANES, LANES)
        s = esrc[o]
        d = edst[o]
        pxs = plsc.load_gather(tpx, [s])
        pxd = plsc.load_gather(tpx, [d])
        pys = plsc.load_gather(tpy, [s])
        pyd = plsc.load_gather(tpy, [d])
        pzs = plsc.load_gather(tpz, [s])
        pzd = plsc.load_gather(tpz, [d])
        gxs = plsc.load_gather(tgx, [s])
        gxd = plsc.load_gather(tgx, [d])
        gys = plsc.load_gather(tgy, [s])
        gyd = plsc.load_gather(tgy, [d])
        gzs = plsc.load_gather(tgz, [s])
        gzd = plsc.load_gather(tgz, [d])
        dx = pxd - pxs
        dy = pyd - pys
        dz = pzd - pzs
        dist2 = dx * dx + dy * dy + dz * dz + EPS
        dive = ((gxd - gxs) * dx + (gyd - gys) * dy + (gzd - gzs) * dz) / dist2
        plsc.addupdate_scatter(adiv, [d], dive)

    pltpu.sync_copy(adiv, sh.at[pl.ds(sid * NPAD, NPAD)])
    plsc.subcore_barrier()
    rdescs = [
        pltpu.async_copy(sh.at[pl.ds(r * NPAD + nbase, SLICE)],
                         red.at[pl.ds(r * SLICE, SLICE)], sem)
        for r in range(NS)
    ]
    for dsc in rdescs:
        dsc.wait()
    _reduce_rows(red, redout)
    pltpu.sync_copy(redout, odiv.at[pl.ds(cid * NPAD + nbase, SLICE)])


def _loss_body(div_ref, cnt_ref, dtp_ref, out_ref):
    d = div_ref[0:1, :] + div_ref[1:2, :]
    c = cnt_ref[0:1, :] + cnt_ref[1:2, :]
    lap = d / (c + EPS)
    diff = dtp_ref[...] - ALPHA_DT * lap
    out_ref[...] = jnp.sum(diff * diff, keepdims=True) * (1.0 / N_NODES)


_loss_call = pl.pallas_call(
    _loss_body,
    out_shape=jax.ShapeDtypeStruct((1, 1), jnp.float32),
)


def kernel(pred, target, x, pos, edge_index, edge_attr):
    xi = jax.lax.bitcast_convert_type(x, jnp.int32).reshape(N_NODES * XF)
    px = pos[:, 0]
    py = pos[:, 1]
    pz = pos[:, 2]
    edges = edge_index.astype(jnp.int32).reshape(2 * N_EDGES)
    dtp = jnp.pad(pred[:, 0], (0, NPAD - N_NODES)).reshape(1, NPAD)

    nx, ny, nz, cnt = _grad_kernel(xi, px, py, pz, edges)
    divp = _div_kernel(nx, ny, nz, cnt, px, py, pz, edges)
    loss = _loss_call(divp.reshape(NC, NPAD), cnt.reshape(NC, NPAD), dtp)
    return loss[0, 0]


# speedup vs baseline: 1.1692x; 1.0224x over previous
import functools

import jax
import jax.numpy as jnp
from jax import lax
from jax.experimental import pallas as pl
from jax.experimental.pallas import tpu as pltpu
from jax.experimental.pallas import tpu_sc as plsc

N_NODES = 10000
N_EDGES = 320000
EPS = 1e-8
ALPHA_DT = 0.6 / (1000.0 * 4186.0) * 1e-05

LANES = 16
NC = 2
NS = 16
NW = NC * NS
NPAD = 10240
EPW = N_EDGES // NW
SLICE = NPAD // NS

_mesh = plsc.VectorSubcoreMesh(
    core_axis_name="c", subcore_axis_name="s", num_cores=NC, num_subcores=NS
)
_sc_params = pltpu.CompilerParams(needs_layout_passes=False)


def _zero_refs(refs):
    z = jnp.zeros((LANES,), jnp.float32)

    @plsc.parallel_loop(0, NPAD // LANES, unroll=4)
    def _(i):
        for r in refs:
            r[pl.ds(i * LANES, LANES)] = z


def _reduce_rows(red, redout):
    @plsc.parallel_loop(0, SLICE // LANES, unroll=2)
    def _(j):
        o = j * LANES
        acc = red[pl.ds(o, LANES)]
        for r in range(1, NS):
            acc = acc + red[pl.ds(r * SLICE + o, LANES)]
        redout[pl.ds(o, LANES)] = acc


XF = 128
XCHUNK = 64
TSL = 640
EOFF = NPAD


@functools.partial(
    pl.kernel,
    out_type=[jax.ShapeDtypeStruct((NC * NPAD,), jnp.float32)] * 4,
    mesh=_mesh,
    scratch_types=[
        pltpu.VMEM((NPAD,), jnp.float32),
        pltpu.VMEM((NPAD,), jnp.float32),
        pltpu.VMEM((NPAD,), jnp.float32),
        pltpu.VMEM((NPAD,), jnp.float32),
        pltpu.VMEM((NPAD,), jnp.float32),
        pltpu.VMEM((NPAD,), jnp.float32),
        pltpu.VMEM((NPAD,), jnp.float32),
        pltpu.VMEM((NPAD,), jnp.float32),
        pltpu.VMEM((2 * NPAD,), jnp.int32),
        pltpu.VMEM((TSL,), jnp.float32),
        pltpu.VMEM((NS * SLICE,), jnp.float32),
        pltpu.VMEM((SLICE,), jnp.float32),
        pltpu.SemaphoreType.DMA,
        pltpu.SemaphoreType.DMA,
        pltpu.VMEM_SHARED((NPAD,), jnp.float32),
        pltpu.VMEM_SHARED((NS * NPAD,), jnp.float32),
    ],
    compiler_params=_sc_params,
)
def _grad_kernel(xi_h, px_h, py_h, pz_h, edges_h,
                 onx, ony, onz, ocnt,
                 tT, tpx, tpy, tpz, ax, ay, az, ac,
                 buf, tcol, red, redout, sem_pos, sem_x, shT, sh):
    cid = lax.axis_index("c")
    sid = lax.axis_index("s")
    wid = cid * NS + sid
    ebase = wid * EPW

    pos_descs = [
        pltpu.async_copy(px_h, tpx.at[pl.ds(0, N_NODES)], sem_pos),
        pltpu.async_copy(py_h, tpy.at[pl.ds(0, N_NODES)], sem_pos),
        pltpu.async_copy(pz_h, tpz.at[pl.ds(0, N_NODES)], sem_pos),
    ]

    col_idx = XF * lax.iota(jnp.int32, LANES) + 3
    nodebase = sid * TSL
    HALF = XCHUNK * XF

    def _extract_chunks(n_chunks, rows_tail):
        sizes = [XCHUNK] * n_chunks + ([rows_tail] if rows_tail else [])

        def _start(c):
            return pltpu.async_copy(
                xi_h.at[pl.ds((nodebase + c * XCHUNK) * XF, sizes[c] * XF)],
                buf.at[pl.ds((c % 2) * HALF, sizes[c] * XF)], sem_x)

        descs = [_start(0)]
        for c in range(len(sizes)):
            descs[c].wait()
            if c + 1 < len(sizes):
                descs.append(_start(c + 1))
            half = (c % 2) * HALF
            for j in range(sizes[c] // LANES):
                v = plsc.load_gather(buf, [col_idx + (half + j * LANES * XF)])
                tcol[pl.ds(c * XCHUNK + j * LANES, LANES)] = plsc.bitcast(
                    v, jnp.float32)

    TAILR = N_NODES - (NS - 1) * TSL

    _zero_refs([ax, ay, az, ac])

    @pl.when(sid < NS - 1)
    def _():
        _extract_chunks(TSL // XCHUNK, 0)
        pltpu.sync_copy(tcol, shT.at[pl.ds(nodebase, TSL)])

    @pl.when(sid == NS - 1)
    def _():
        _extract_chunks(TAILR // XCHUNK, TAILR % XCHUNK)
        pltpu.sync_copy(tcol.at[pl.ds(0, TAILR)],
                        shT.at[pl.ds(nodebase, TAILR)])

    edge_descs = [
        pltpu.async_copy(edges_h.at[pl.ds(ebase, EPW)],
                         buf.at[pl.ds(0, EPW)], sem_pos),
        pltpu.async_copy(edges_h.at[pl.ds(N_EDGES + ebase, EPW)],
                         buf.at[pl.ds(EOFF, EPW)], sem_pos),
    ]
    for dsc in pos_descs:
        dsc.wait()
    for dsc in edge_descs:
        dsc.wait()
    plsc.subcore_barrier()
    pltpu.sync_copy(shT, tT)
    ones = jnp.full((LANES,), 1.0, jnp.float32)

    @plsc.parallel_loop(0, EPW // LANES, unroll=4)
    def _(i):
        o = pl.ds(i * LANES, LANES)
        s = buf[o]
        d = buf[pl.ds(EOFF + i * LANES, LANES)]
        ts = plsc.load_gather(tT, [s])
        td = plsc.load_gather(tT, [d])
        pxs = plsc.load_gather(tpx, [s])
        pxd = plsc.load_gather(tpx, [d])
        pys = plsc.load_gather(tpy, [s])
        pyd = plsc.load_gather(tpy, [d])
        pzs = plsc.load_gather(tpz, [s])
        pzd = plsc.load_gather(tpz, [d])
        dx = pxd - pxs
        dy = pyd - pys
        dz = pzd - pzs
        dist2 = dx * dx + dy * dy + dz * dz + EPS
        w = (td - ts) / dist2
        plsc.addupdate_scatter(ax, [d], w * dx)
        plsc.addupdate_scatter(ay, [d], w * dy)
        plsc.addupdate_scatter(az, [d], w * dz)
        plsc.addupdate_scatter(ac, [d], ones)

    nbase = sid * SLICE
    for k, (acc, out) in enumerate([(ax, onx), (ay, ony), (az, onz), (ac, ocnt)]):
        if k > 0:
            plsc.subcore_barrier()
        pltpu.sync_copy(acc, sh.at[pl.ds(sid * NPAD, NPAD)])
        plsc.subcore_barrier()
        descs = [
            pltpu.async_copy(sh.at[pl.ds(r * NPAD + nbase, SLICE)],
                             red.at[pl.ds(r * SLICE, SLICE)], sem_x)
            for r in range(NS)
        ]
        for dsc in descs:
            dsc.wait()
        _reduce_rows(red, redout)
        pltpu.sync_copy(redout, out.at[pl.ds(cid * NPAD + nbase, SLICE)])


@functools.partial(
    pl.kernel,
    out_type=jax.ShapeDtypeStruct((NC * NPAD,), jnp.float32),
    mesh=_mesh,
    scratch_types=[
        pltpu.VMEM((NPAD,), jnp.float32),
        pltpu.VMEM((NPAD,), jnp.float32),
        pltpu.VMEM((NPAD,), jnp.float32),
        pltpu.VMEM((NPAD,), jnp.float32),
        pltpu.VMEM((NPAD,), jnp.float32),
        pltpu.VMEM((NPAD,), jnp.float32),
        pltpu.VMEM((NPAD,), jnp.float32),
        pltpu.VMEM((EPW,), jnp.int32),
        pltpu.VMEM((EPW,), jnp.int32),
        pltpu.VMEM((NS * SLICE,), jnp.float32),
        pltpu.VMEM((SLICE,), jnp.float32),
        pltpu.VMEM((NC * SLICE,), jnp.float32),
        pltpu.VMEM((SLICE,), jnp.float32),
        pltpu.SemaphoreType.DMA,
        pltpu.VMEM_SHARED((3 * NPAD,), jnp.float32),
        pltpu.VMEM_SHARED((NS * NPAD,), jnp.float32),
    ],
    compiler_params=_sc_params,
)
def _div_kernel(pnx, pny, pnz, pcnt, px_h, py_h, pz_h, edges_h,
                odiv,
                tpx, tpy, tpz, tgx, tgy, tgz, adiv,
                esrc, edst, red, redout, pbuf, ccnt, sem, shg, sh):
    cid = lax.axis_index("c")
    sid = lax.axis_index("s")
    wid = cid * NS + sid
    ebase = wid * EPW
    nbase = sid * SLICE

    stage_descs = [
        pltpu.async_copy(px_h, tpx.at[pl.ds(0, N_NODES)], sem),
        pltpu.async_copy(py_h, tpy.at[pl.ds(0, N_NODES)], sem),
        pltpu.async_copy(pz_h, tpz.at[pl.ds(0, N_NODES)], sem),
        pltpu.async_copy(edges_h.at[pl.ds(ebase, EPW)], esrc, sem),
        pltpu.async_copy(edges_h.at[pl.ds(N_EDGES + ebase, EPW)], edst, sem),
    ]

    pltpu.sync_copy(pcnt.at[pl.ds(nbase, SLICE)], pbuf.at[pl.ds(0, SLICE)])
    pltpu.sync_copy(pcnt.at[pl.ds(NPAD + nbase, SLICE)],
                    pbuf.at[pl.ds(SLICE, SLICE)])

    def cnt_body(j, _):
        o = j * LANES
        ccnt[pl.ds(o, LANES)] = (
            pbuf[pl.ds(o, LANES)] + pbuf[pl.ds(SLICE + o, LANES)] + EPS
        )
        return 0

    lax.fori_loop(0, SLICE // LANES, cnt_body, 0)

    for k, pn in enumerate([pnx, pny, pnz]):
        pltpu.sync_copy(pn.at[pl.ds(nbase, SLICE)], pbuf.at[pl.ds(0, SLICE)])
        pltpu.sync_copy(pn.at[pl.ds(NPAD + nbase, SLICE)],
                        pbuf.at[pl.ds(SLICE, SLICE)])

        def g_body(j, _):
            o = j * LANES
            redout[pl.ds(o, LANES)] = (
                pbuf[pl.ds(o, LANES)] + pbuf[pl.ds(SLICE + o, LANES)]
            ) / ccnt[pl.ds(o, LANES)]
            return 0

        lax.fori_loop(0, SLICE // LANES, g_body, 0)
        pltpu.sync_copy(redout, shg.at[pl.ds(k * NPAD + nbase, SLICE)])

    plsc.subcore_barrier()
    gdescs = [
        pltpu.async_copy(shg.at[pl.ds(0, NPAD)], tgx, sem),
        pltpu.async_copy(shg.at[pl.ds(NPAD, NPAD)], tgy, sem),
        pltpu.async_copy(shg.at[pl.ds(2 * NPAD, NPAD)], tgz, sem),
    ]
    _zero_refs([adiv])
    for dsc in stage_descs:
        dsc.wait()
    for dsc in gdescs:
        dsc.wait()

    @plsc.parallel_loop(0, EPW // LANES, unroll=4)
    def _(i):
        o = pl.ds(i * LANES, LANES)
        s = esrc[o]
        d = edst[o]
        pxs = plsc.load_gather(tpx, [s])
        pxd = plsc.load_gather(tpx, [d])
        pys = plsc.load_gather(tpy, [s])
        pyd = plsc.load_gather(tpy, [d])
        pzs = plsc.load_gather(tpz, [s])
        pzd = plsc.load_gather(tpz, [d])
        gxs = plsc.load_gather(tgx, [s])
        gxd = plsc.load_gather(tgx, [d])
        gys = plsc.load_gather(tgy, [s])
        gyd = plsc.load_gather(tgy, [d])
        gzs = plsc.load_gather(tgz, [s])
        gzd = plsc.load_gather(tgz, [d])
        dx = pxd - pxs
        dy = pyd - pys
        dz = pzd - pzs
        dist2 = dx * dx + dy * dy + dz * dz + EPS
        dive = ((gxd - gxs) * dx + (gyd - gys) * dy + (gzd - gzs) * dz) / dist2
        plsc.addupdate_scatter(adiv, [d], dive)

    pltpu.sync_copy(adiv, sh.at[pl.ds(sid * NPAD, NPAD)])
    plsc.subcore_barrier()
    rdescs = [
        pltpu.async_copy(sh.at[pl.ds(r * NPAD + nbase, SLICE)],
                         red.at[pl.ds(r * SLICE, SLICE)], sem)
        for r in range(NS)
    ]
    for dsc in rdescs:
        dsc.wait()
    _reduce_rows(red, redout)
    pltpu.sync_copy(redout, odiv.at[pl.ds(cid * NPAD + nbase, SLICE)])


def _loss_body(div_ref, cnt_ref, dtp_ref, out_ref):
    d = div_ref[0:1, :] + div_ref[1:2, :]
    c = cnt_ref[0:1, :] + cnt_ref[1:2, :]
    lap = d / (c + EPS)
    diff = dtp_ref[...] - ALPHA_DT * lap
    out_ref[...] = jnp.sum(diff * diff, keepdims=True) * (1.0 / N_NODES)


_loss_call = pl.pallas_call(
    _loss_body,
    out_shape=jax.ShapeDtypeStruct((1, 1), jnp.float32),
)


def kernel(pred, target, x, pos, edge_index, edge_attr):
    xi = jax.lax.bitcast_convert_type(x, jnp.int32).reshape(N_NODES * XF)
    px = pos[:, 0]
    py = pos[:, 1]
    pz = pos[:, 2]
    edges = edge_index.astype(jnp.int32).reshape(2 * N_EDGES)
    dtp = jnp.pad(pred[:, 0], (0, NPAD - N_NODES)).reshape(1, NPAD)

    nx, ny, nz, cnt = _grad_kernel(xi, px, py, pz, edges)
    divp = _div_kernel(nx, ny, nz, cnt, px, py, pz, edges)
    loss = _loss_call(divp.reshape(NC, NPAD), cnt.reshape(NC, NPAD), dtp)
    return loss[0, 0]
